# baseline (speedup 1.0000x reference)
import functools

import jax
import jax.numpy as jnp
from jax import lax
from jax.experimental import pallas as pl
from jax.experimental.pallas import tpu as pltpu
from jax.experimental.pallas import tpu_sc as plsc

_LANES = 16
_NC = 2
_NS = 16
_NW = _NC * _NS


def _make_sc_pool(B, LP, EMB):
    bags_per_w = B // _NW
    chunk = 8
    nchunks = bags_per_w // chunk
    unroll = 8
    nsub = EMB // _LANES

    mesh = plsc.VectorSubcoreMesh(core_axis_name="c", subcore_axis_name="s")

    @functools.partial(
        pl.kernel,
        mesh=mesh,
        compiler_params=pltpu.CompilerParams(use_tc_tiling_on_sc=False),
        out_type=jax.ShapeDtypeStruct((B, EMB), jnp.float32),
        scratch_types=[
            pltpu.VMEM((chunk, LP), jnp.int32),
            pltpu.VMEM((chunk, LP, EMB), jnp.float32),
            pltpu.VMEM((bags_per_w, EMB), jnp.float32),
            pltpu.SemaphoreType.DMA,
        ],
    )
    def sc_pool(data_hbm, table_hbm, out_hbm, idx_v, rows_v, pooled_v, sem):
        wid = lax.axis_index("s") * _NC + lax.axis_index("c")
        bag0 = wid * bags_per_w

        def chunk_body(g, carry):
            row0 = bag0 + g * chunk
            pltpu.sync_copy(data_hbm.at[pl.ds(row0, chunk), :], idx_v)
            copies = [
                pltpu.async_copy(table_hbm.at[idx_v.at[i]], rows_v.at[i], sem)
                for i in range(chunk)
            ]
            for c in copies:
                c.wait()
            for i in range(chunk):
                def accum(it, accs):
                    out = list(accs)
                    for u in range(unroll):
                        tok = it * unroll + u
                        for j in range(nsub):
                            out[j] = out[j] + rows_v[i, tok, pl.ds(j * _LANES, _LANES)]
                    return tuple(out)

                zero = jnp.zeros((_LANES,), jnp.float32)
                accs = lax.fori_loop(0, LP // unroll, accum, (zero,) * nsub)
                for j in range(nsub):
                    pooled_v[g * chunk + i, pl.ds(j * _LANES, _LANES)] = accs[j]
            return carry

        lax.fori_loop(0, nchunks, chunk_body, 0)
        pltpu.sync_copy(pooled_v, out_hbm.at[pl.ds(bag0, bags_per_w), :])

    return sc_pool


def _finalize(pooled, lenf, wt, b2):
    B, EMB = pooled.shape
    NCLS = wt.shape[1]

    def body(p_ref, l_ref, w_ref, b_ref, o_ref):
        x = p_ref[...] / l_ref[...]
        o_ref[...] = (
            jnp.dot(x, w_ref[...], preferred_element_type=jnp.float32) + b_ref[...]
        )

    return pl.pallas_call(
        body,
        out_shape=jax.ShapeDtypeStruct((B, NCLS), jnp.float32),
    )(pooled, lenf, wt, b2)


def kernel(data, length, embed_table, W, b):
    B, L = data.shape
    EMB = embed_table.shape[1]
    NCLS = W.shape[0]
    LP = 56

    data_pad = jnp.concatenate(
        [data, jnp.zeros((B, LP - L), jnp.int32)], axis=1
    )
    pooled = _make_sc_pool(B, LP, EMB)(data_pad, embed_table)
    lenf = length.astype(jnp.float32).reshape(B, 1)
    return _finalize(pooled, lenf, W.T, b.reshape(1, NCLS))
